# baseline (speedup 1.0000x reference)
import functools

import jax
import jax.numpy as jnp
from jax import lax
from jax.experimental import pallas as pl
from jax.experimental.pallas import tpu as pltpu
from jax.experimental.pallas import tpu_sc as plsc

NC = 2
NS = 16
NW = NC * NS
B = 100
H = 128
ZR = 125


def _node_proj_body(h_ref, w1a_ref, w1b_ref, hpa_ref, hpb_ref):
    hblk = h_ref[...]
    hpa_ref[...] = jnp.dot(hblk, w1a_ref[...],
                           preferred_element_type=jnp.float32).astype(jnp.bfloat16)
    hpb_ref[...] = jnp.dot(hblk, w1b_ref[...],
                           preferred_element_type=jnp.float32).astype(jnp.bfloat16)


def _node_proj(h, w1a, w1b):
    n = h.shape[0]
    blk = 2000
    return pl.pallas_call(
        _node_proj_body,
        out_shape=(jax.ShapeDtypeStruct((n, H), jnp.bfloat16),
                   jax.ShapeDtypeStruct((n, H), jnp.bfloat16)),
        grid=(n // blk,),
        in_specs=[pl.BlockSpec((blk, H), lambda i: (i, 0)),
                  pl.BlockSpec((H, H), lambda i: (0, 0)),
                  pl.BlockSpec((H, H), lambda i: (0, 0))],
        out_specs=(pl.BlockSpec((blk, H), lambda i: (i, 0)),
                   pl.BlockSpec((blk, H), lambda i: (i, 0))),
    )(h, w1a, w1b)


def _make_gather(e_total):
    rows = e_total // B
    rpw = rows // NW
    mesh = plsc.VectorSubcoreMesh(core_axis_name="c", subcore_axis_name="s")

    @functools.partial(
        pl.kernel,
        out_type=jax.ShapeDtypeStruct((e_total, H), jnp.bfloat16),
        mesh=mesh,
        compiler_params=pltpu.CompilerParams(use_tc_tiling_on_sc=False),
        scratch_types=[
            pltpu.VMEM((rpw, B), jnp.int32),
            pltpu.VMEM((rpw, B), jnp.int32),
            pltpu.VMEM((2, B, H), jnp.bfloat16),
            pltpu.VMEM((2, B, H), jnp.bfloat16),
            pltpu.SemaphoreType.DMA,
            pltpu.SemaphoreType.DMA,
            pltpu.SemaphoreType.DMA,
            pltpu.SemaphoreType.DMA,
        ],
    )
    def gather_k(hpa, hpb, src2, dst2, out, idxs, idxd, bufa, bufb,
                 sa0, sa1, sb0, sb1):
        cid = lax.axis_index("c")
        sid = lax.axis_index("s")
        wid = sid * NC + cid
        r0 = wid * rpw
        pltpu.sync_copy(src2.at[pl.ds(r0, rpw)], idxs)
        pltpu.sync_copy(dst2.at[pl.ds(r0, rpw)], idxd)
        sa = (sa0, sa1)
        sb = (sb0, sb1)

        def issue(ch, slot):
            pltpu.async_copy(hpa.at[idxs.at[ch]], bufa.at[slot], sa[slot])
            pltpu.async_copy(hpb.at[idxd.at[ch]], bufb.at[slot], sb[slot])

        def wait(slot):
            pltpu.make_async_copy(hpa.at[idxs.at[0]], bufa.at[slot], sa[slot]).wait()
            pltpu.make_async_copy(hpb.at[idxd.at[0]], bufb.at[slot], sb[slot]).wait()

        def process(slot, ch):
            def row(r, _):
                for k in range(H // 32):
                    s = pl.ds(k * 32, 32)
                    bufa[slot, r, s] = bufa[slot, r, s] + bufb[slot, r, s]
                return 0
            lax.fori_loop(0, B, row, 0)
            off = (r0 + ch) * B
            pltpu.sync_copy(bufa.at[slot], out.at[pl.ds(off, B)])

        issue(0, 0)

        def body2(i, _):
            ch0 = 2 * i
            wait(0)
            issue(ch0 + 1, 1)
            process(0, ch0)
            wait(1)

            @pl.when(ch0 + 2 < rpw)
            def _():
                issue(ch0 + 2, 0)

            process(1, ch0 + 1)
            return 0

        lax.fori_loop(0, rpw // 2, body2, 0)

    return gather_k


def _edge_body(g_ref, e_ref, w1c_ref, b1_ref, w2_ref, b2_ref, wet_ref, be_ref,
               en_ref, mw_ref):
    eblk = e_ref[...]
    x = (g_ref[...].astype(jnp.float32)
         + jnp.dot(eblk, w1c_ref[...], preferred_element_type=jnp.float32)
         + b1_ref[...])
    hid = x * jax.nn.sigmoid(x)
    msg = jnp.dot(hid, w2_ref[...], preferred_element_type=jnp.float32) + b2_ref[...]
    en = eblk + msg
    w = jax.nn.sigmoid(jnp.sum(en * wet_ref[...], axis=1, keepdims=True) + be_ref[...])
    en_ref[...] = en
    mw_ref[...] = msg * w


def _edge_mlp(g, e, w1c, b1, w2, b2, wet, be):
    e_total = e.shape[0]
    blk = 2000
    wspec = pl.BlockSpec((H, H), lambda i: (0, 0))
    bspec = pl.BlockSpec((1, H), lambda i: (0, 0))
    return pl.pallas_call(
        _edge_body,
        out_shape=(jax.ShapeDtypeStruct((e_total, H), jnp.float32),
                   jax.ShapeDtypeStruct((e_total, H), jnp.float32)),
        grid=(e_total // blk,),
        in_specs=[pl.BlockSpec((blk, H), lambda i: (i, 0)),
                  pl.BlockSpec((blk, H), lambda i: (i, 0)),
                  wspec, bspec, wspec, bspec, bspec,
                  pl.BlockSpec((1, 1), lambda i: (0, 0))],
        out_specs=(pl.BlockSpec((blk, H), lambda i: (i, 0)),
                   pl.BlockSpec((blk, H), lambda i: (i, 0))),
    )(g, e, w1c, b1, w2, b2, wet, be)


def _make_scatter(e_total, n):
    rows = e_total // B
    rpw = rows // NW
    nps = n // NS
    mesh = plsc.VectorSubcoreMesh(core_axis_name="c", subcore_axis_name="s")
    f32 = jnp.float32

    @functools.partial(
        pl.kernel,
        out_type=(jax.ShapeDtypeStruct((n, H), f32),
                  jax.ShapeDtypeStruct((n, H), f32),
                  jax.ShapeDtypeStruct((n, 16), f32),
                  jax.ShapeDtypeStruct((n, 16), f32)),
        mesh=mesh,
        compiler_params=pltpu.CompilerParams(use_tc_tiling_on_sc=False),
        scratch_types=[
            pltpu.VMEM((rpw, B), jnp.int32),
            pltpu.VMEM((2, B, H), f32),
            pltpu.VMEM((B, 16), f32),
            pltpu.VMEM((B, 16), f32),
            pltpu.VMEM_SHARED((n, H), f32),
            pltpu.VMEM_SHARED((n, 16), f32),
            pltpu.SemaphoreType.DMA,
            pltpu.SemaphoreType.DMA,
        ],
    )
    def scatter_k(mw, dst2, agg0, agg1, deg0, deg1,
                  dstv, val, ones_v, dstage, aggs, degs, sv0, sv1):
        cid = lax.axis_index("c")
        sid = lax.axis_index("s")
        wid = sid * NC + cid
        r0 = wid * rpw
        nblk = n // B
        bpt = (nblk + NS - 1) // NS

        zero16 = jnp.zeros((16,), f32)
        one16 = jnp.ones((16,), f32)

        def zval(r, _):
            for k in range(H // 16):
                val[0, r, pl.ds(k * 16, 16)] = zero16
            return 0
        lax.fori_loop(0, B, zval, 0)

        def zsmall(r, _):
            dstage[r, pl.ds(0, 16)] = zero16
            ones_v[r, pl.ds(0, 16)] = one16
            return 0
        lax.fori_loop(0, B, zsmall, 0)

        def zblk(k, _):
            blk = k * NS + sid

            @pl.when(blk < nblk)
            def _():
                sl = pl.ds(blk * B, B)
                pltpu.sync_copy(val.at[0], aggs.at[sl])
                pltpu.sync_copy(dstage, degs.at[sl])
            return 0
        lax.fori_loop(0, bpt, zblk, 0)
        plsc.subcore_barrier()

        pltpu.sync_copy(dst2.at[pl.ds(r0, rpw)], dstv)
        sv = (sv0, sv1)

        def issue(ch, slot):
            off = (r0 + ch) * B
            pltpu.async_copy(mw.at[pl.ds(off, B)], val.at[slot], sv[slot])

        def wait(slot):
            pltpu.make_async_copy(mw.at[pl.ds(0, B)], val.at[slot], sv[slot]).wait()

        def process(slot, ch):
            pltpu.sync_copy(val.at[slot], aggs.at[dstv.at[ch]], add=True)
            pltpu.sync_copy(ones_v, degs.at[dstv.at[ch]], add=True)

        issue(0, 0)

        def body2(i, _):
            ch0 = 2 * i
            wait(0)
            issue(ch0 + 1, 1)
            process(0, ch0)
            wait(1)

            @pl.when(ch0 + 2 < rpw)
            def _():
                issue(ch0 + 2, 0)

            process(1, ch0 + 1)
            return 0

        lax.fori_loop(0, rpw // 2, body2, 0)
        plsc.subcore_barrier()

        def wblk(k, _):
            blk = k * NS + sid

            @pl.when(blk < nblk)
            def _():
                sl = pl.ds(blk * B, B)
                pltpu.sync_copy(aggs.at[sl], val.at[0])
                pltpu.sync_copy(degs.at[sl], dstage)

                @pl.when(cid == 0)
                def _():
                    pltpu.sync_copy(val.at[0], agg0.at[sl])
                    pltpu.sync_copy(dstage, deg0.at[sl])

                @pl.when(cid == 1)
                def _():
                    pltpu.sync_copy(val.at[0], agg1.at[sl])
                    pltpu.sync_copy(dstage, deg1.at[sl])
            return 0
        lax.fori_loop(0, bpt, wblk, 0)

    return scatter_k


def _update_body(h_ref, p0_ref, p1_ref, d0_ref, d1_ref, u1a_ref, u1b_ref,
                 ub1_ref, u2_ref, ub2_ref, out_ref):
    s = p0_ref[...] + p1_ref[...]
    d = d0_ref[...][:, 0:1] + d1_ref[...][:, 0:1]
    agg = s / (d + 1e-6)
    x = (jnp.dot(h_ref[...], u1a_ref[...], preferred_element_type=jnp.float32)
         + jnp.dot(agg, u1b_ref[...], preferred_element_type=jnp.float32)
         + ub1_ref[...])
    hid = x * jax.nn.sigmoid(x)
    out_ref[...] = (jnp.dot(hid, u2_ref[...], preferred_element_type=jnp.float32)
                    + ub2_ref[...])


def _update_mlp(h, p0, p1, d0, d1, u1a, u1b, ub1, u2, ub2):
    n = h.shape[0]
    blk = 2000
    wspec = pl.BlockSpec((H, H), lambda i: (0, 0))
    bspec = pl.BlockSpec((1, H), lambda i: (0, 0))
    nspec = pl.BlockSpec((blk, H), lambda i: (i, 0))
    dspec = pl.BlockSpec((blk, 16), lambda i: (i, 0))
    return pl.pallas_call(
        _update_body,
        out_shape=jax.ShapeDtypeStruct((n, H), jnp.float32),
        grid=(n // blk,),
        in_specs=[nspec, nspec, nspec, dspec, dspec,
                  wspec, wspec, bspec, wspec, bspec],
        out_specs=nspec,
    )(h, p0, p1, d0, d1, u1a, u1b, ub1, u2, ub2)


def kernel(edge_index, h, e, W1, b1, W2, b2, We, be, U1, ub1, U2, ub2):
    n, hdim = h.shape
    src2 = edge_index[0].reshape(-1, B)
    dst2 = edge_index[1].reshape(-1, B)
    w1a, w1b, w1c = W1[:hdim], W1[hdim:2 * hdim], W1[2 * hdim:]

    hpa, hpb = _node_proj(h, w1a, w1b)
    g = _make_gather(e.shape[0])(hpa, hpb, src2, dst2)
    en, mw = _edge_mlp(g, e, w1c, b1.reshape(1, -1), W2, b2.reshape(1, -1),
                       We.reshape(1, -1), be.reshape(1, 1))
    agg0, agg1, deg0, deg1 = _make_scatter(e.shape[0], n)(mw, dst2)
    h_new = _update_mlp(h, agg0, agg1, deg0, deg1, U1[:hdim], U1[hdim:],
                        ub1.reshape(1, -1), U2, ub2.reshape(1, -1))
    return (h_new, en)


# speedup vs baseline: 1.2946x; 1.2946x over previous
import functools

import jax
import jax.numpy as jnp
from jax import lax
from jax.experimental import pallas as pl
from jax.experimental.pallas import tpu as pltpu
from jax.experimental.pallas import tpu_sc as plsc

NC = 2
NS = 16
NW = NC * NS
B = 100
H = 128
ZR = 125


def _node_proj_body(h_ref, w1a_ref, w1b_ref, hpa_ref, hpb_ref):
    hblk = h_ref[...]
    hpa_ref[...] = jnp.dot(hblk, w1a_ref[...],
                           preferred_element_type=jnp.float32).astype(jnp.bfloat16)
    hpb_ref[...] = jnp.dot(hblk, w1b_ref[...],
                           preferred_element_type=jnp.float32).astype(jnp.bfloat16)


def _node_proj(h, w1a, w1b):
    n = h.shape[0]
    blk = 2000
    return pl.pallas_call(
        _node_proj_body,
        out_shape=(jax.ShapeDtypeStruct((n, H), jnp.bfloat16),
                   jax.ShapeDtypeStruct((n, H), jnp.bfloat16)),
        grid=(n // blk,),
        in_specs=[pl.BlockSpec((blk, H), lambda i: (i, 0)),
                  pl.BlockSpec((H, H), lambda i: (0, 0)),
                  pl.BlockSpec((H, H), lambda i: (0, 0))],
        out_specs=(pl.BlockSpec((blk, H), lambda i: (i, 0)),
                   pl.BlockSpec((blk, H), lambda i: (i, 0))),
    )(h, w1a, w1b)


def _make_gather(e_total):
    rows = e_total // B
    rpw = rows // NW
    mesh = plsc.VectorSubcoreMesh(core_axis_name="c", subcore_axis_name="s")

    @functools.partial(
        pl.kernel,
        out_type=jax.ShapeDtypeStruct((e_total, H // 2), jnp.int32),
        mesh=mesh,
        compiler_params=pltpu.CompilerParams(use_tc_tiling_on_sc=False,
                                             needs_layout_passes=False),
        scratch_types=[
            pltpu.VMEM((rpw, B), jnp.int32),
            pltpu.VMEM((rpw, B), jnp.int32),
            pltpu.VMEM((2, B, H // 2), jnp.int32),
            pltpu.VMEM((2, B, H // 2), jnp.int32),
            pltpu.SemaphoreType.DMA,
            pltpu.SemaphoreType.DMA,
            pltpu.SemaphoreType.DMA,
            pltpu.SemaphoreType.DMA,
        ],
    )
    def gather_k(hpa, hpb, src2, dst2, out, idxs, idxd, bufa, bufb,
                 sa0, sa1, sb0, sb1):
        cid = lax.axis_index("c")
        sid = lax.axis_index("s")
        wid = sid * NC + cid
        r0 = wid * rpw
        pltpu.sync_copy(src2.at[pl.ds(r0, rpw)], idxs)
        pltpu.sync_copy(dst2.at[pl.ds(r0, rpw)], idxd)
        sa = (sa0, sa1)
        sb = (sb0, sb1)

        def issue(ch, slot):
            pltpu.async_copy(hpa.at[idxs.at[ch]], bufa.at[slot], sa[slot])
            pltpu.async_copy(hpb.at[idxd.at[ch]], bufb.at[slot], sb[slot])

        def wait(slot):
            pltpu.make_async_copy(hpa.at[idxs.at[0]], bufa.at[slot], sa[slot]).wait()
            pltpu.make_async_copy(hpb.at[idxd.at[0]], bufb.at[slot], sb[slot]).wait()

        def process(slot, ch):
            def row(r, _):
                for k in range(H // 32):
                    s = pl.ds(k * 16, 16)
                    a = plsc.bitcast(bufa[slot, r, s], jnp.bfloat16)
                    b = plsc.bitcast(bufb[slot, r, s], jnp.bfloat16)
                    bufa[slot, r, s] = plsc.bitcast(a + b, jnp.int32)
                return 0
            lax.fori_loop(0, B, row, 0)
            off = (r0 + ch) * B
            pltpu.sync_copy(bufa.at[slot], out.at[pl.ds(off, B)])

        issue(0, 0)

        def body2(i, _):
            ch0 = 2 * i
            wait(0)
            issue(ch0 + 1, 1)
            process(0, ch0)
            wait(1)

            @pl.when(ch0 + 2 < rpw)
            def _():
                issue(ch0 + 2, 0)

            process(1, ch0 + 1)
            return 0

        lax.fori_loop(0, rpw // 2, body2, 0)

    return gather_k


def _edge_body(g_ref, e_ref, w1c_ref, b1_ref, w2_ref, b2_ref, wet_ref, be_ref,
               en_ref, mw_ref):
    eblk = e_ref[...]
    gw = g_ref[...]
    glo = jax.lax.bitcast_convert_type(gw << 16, jnp.float32)
    ghi = jax.lax.bitcast_convert_type((gw >> 16) << 16, jnp.float32)
    g128 = jnp.concatenate([glo, ghi], axis=-1)
    x = (g128
         + jnp.dot(eblk, w1c_ref[...], preferred_element_type=jnp.float32)
         + b1_ref[...])
    hid = x * jax.nn.sigmoid(x)
    msg = jnp.dot(hid, w2_ref[...], preferred_element_type=jnp.float32) + b2_ref[...]
    en = eblk + msg
    w = jax.nn.sigmoid(jnp.sum(en * wet_ref[...], axis=1, keepdims=True) + be_ref[...])
    en_ref[...] = en
    mw_ref[...] = msg * w


def _edge_mlp(g, e, w1c, b1, w2, b2, wet, be):
    e_total = e.shape[0]
    blk = 2000
    wspec = pl.BlockSpec((H, H), lambda i: (0, 0))
    bspec = pl.BlockSpec((1, H), lambda i: (0, 0))
    return pl.pallas_call(
        _edge_body,
        out_shape=(jax.ShapeDtypeStruct((e_total, H), jnp.float32),
                   jax.ShapeDtypeStruct((e_total, H), jnp.float32)),
        grid=(e_total // blk,),
        in_specs=[pl.BlockSpec((blk, H // 2), lambda i: (i, 0)),
                  pl.BlockSpec((blk, H), lambda i: (i, 0)),
                  wspec, bspec, wspec, bspec, bspec,
                  pl.BlockSpec((1, 1), lambda i: (0, 0))],
        out_specs=(pl.BlockSpec((blk, H), lambda i: (i, 0)),
                   pl.BlockSpec((blk, H), lambda i: (i, 0))),
    )(g, e, w1c, b1, w2, b2, wet, be)


def _make_scatter(e_total, n):
    rows = e_total // B
    rpw = rows // NW
    nps = n // NS
    mesh = plsc.VectorSubcoreMesh(core_axis_name="c", subcore_axis_name="s")
    f32 = jnp.float32

    @functools.partial(
        pl.kernel,
        out_type=(jax.ShapeDtypeStruct((n, H), f32),
                  jax.ShapeDtypeStruct((n, H), f32),
                  jax.ShapeDtypeStruct((n, 16), f32),
                  jax.ShapeDtypeStruct((n, 16), f32)),
        mesh=mesh,
        compiler_params=pltpu.CompilerParams(use_tc_tiling_on_sc=False),
        scratch_types=[
            pltpu.VMEM((rpw, B), jnp.int32),
            pltpu.VMEM((2, B, H), f32),
            pltpu.VMEM((B, 16), f32),
            pltpu.VMEM((B, 16), f32),
            pltpu.VMEM_SHARED((n, H), f32),
            pltpu.VMEM_SHARED((n, 16), f32),
            pltpu.SemaphoreType.DMA,
            pltpu.SemaphoreType.DMA,
        ],
    )
    def scatter_k(mw, dst2, agg0, agg1, deg0, deg1,
                  dstv, val, ones_v, dstage, aggs, degs, sv0, sv1):
        cid = lax.axis_index("c")
        sid = lax.axis_index("s")
        wid = sid * NC + cid
        r0 = wid * rpw
        nblk = n // B
        bpt = (nblk + NS - 1) // NS

        zero16 = jnp.zeros((16,), f32)
        one16 = jnp.ones((16,), f32)

        def zval(r, _):
            for k in range(H // 16):
                val[0, r, pl.ds(k * 16, 16)] = zero16
            return 0
        lax.fori_loop(0, B, zval, 0)

        def zsmall(r, _):
            dstage[r, pl.ds(0, 16)] = zero16
            ones_v[r, pl.ds(0, 16)] = one16
            return 0
        lax.fori_loop(0, B, zsmall, 0)

        def zblk(k, _):
            blk = k * NS + sid

            @pl.when(blk < nblk)
            def _():
                sl = pl.ds(blk * B, B)
                pltpu.sync_copy(val.at[0], aggs.at[sl])
                pltpu.sync_copy(dstage, degs.at[sl])
            return 0
        lax.fori_loop(0, bpt, zblk, 0)
        plsc.subcore_barrier()

        pltpu.sync_copy(dst2.at[pl.ds(r0, rpw)], dstv)
        sv = (sv0, sv1)

        def issue(ch, slot):
            off = (r0 + ch) * B
            pltpu.async_copy(mw.at[pl.ds(off, B)], val.at[slot], sv[slot])

        def wait(slot):
            pltpu.make_async_copy(mw.at[pl.ds(0, B)], val.at[slot], sv[slot]).wait()

        def process(slot, ch):
            pltpu.sync_copy(val.at[slot], aggs.at[dstv.at[ch]], add=True)
            pltpu.sync_copy(ones_v, degs.at[dstv.at[ch]], add=True)

        issue(0, 0)

        def body2(i, _):
            ch0 = 2 * i
            wait(0)
            issue(ch0 + 1, 1)
            process(0, ch0)
            wait(1)

            @pl.when(ch0 + 2 < rpw)
            def _():
                issue(ch0 + 2, 0)

            process(1, ch0 + 1)
            return 0

        lax.fori_loop(0, rpw // 2, body2, 0)
        plsc.subcore_barrier()

        def wblk(k, _):
            blk = k * NS + sid

            @pl.when(blk < nblk)
            def _():
                sl = pl.ds(blk * B, B)
                pltpu.sync_copy(aggs.at[sl], val.at[0])
                pltpu.sync_copy(degs.at[sl], dstage)

                @pl.when(cid == 0)
                def _():
                    pltpu.sync_copy(val.at[0], agg0.at[sl])
                    pltpu.sync_copy(dstage, deg0.at[sl])

                @pl.when(cid == 1)
                def _():
                    pltpu.sync_copy(val.at[0], agg1.at[sl])
                    pltpu.sync_copy(dstage, deg1.at[sl])
            return 0
        lax.fori_loop(0, bpt, wblk, 0)

    return scatter_k


def _update_body(h_ref, p0_ref, p1_ref, d0_ref, d1_ref, u1a_ref, u1b_ref,
                 ub1_ref, u2_ref, ub2_ref, out_ref):
    s = p0_ref[...] + p1_ref[...]
    d = d0_ref[...][:, 0:1] + d1_ref[...][:, 0:1]
    agg = s / (d + 1e-6)
    x = (jnp.dot(h_ref[...], u1a_ref[...], preferred_element_type=jnp.float32)
         + jnp.dot(agg, u1b_ref[...], preferred_element_type=jnp.float32)
         + ub1_ref[...])
    hid = x * jax.nn.sigmoid(x)
    out_ref[...] = (jnp.dot(hid, u2_ref[...], preferred_element_type=jnp.float32)
                    + ub2_ref[...])


def _update_mlp(h, p0, p1, d0, d1, u1a, u1b, ub1, u2, ub2):
    n = h.shape[0]
    blk = 2000
    wspec = pl.BlockSpec((H, H), lambda i: (0, 0))
    bspec = pl.BlockSpec((1, H), lambda i: (0, 0))
    nspec = pl.BlockSpec((blk, H), lambda i: (i, 0))
    dspec = pl.BlockSpec((blk, 16), lambda i: (i, 0))
    return pl.pallas_call(
        _update_body,
        out_shape=jax.ShapeDtypeStruct((n, H), jnp.float32),
        grid=(n // blk,),
        in_specs=[nspec, nspec, nspec, dspec, dspec,
                  wspec, wspec, bspec, wspec, bspec],
        out_specs=nspec,
    )(h, p0, p1, d0, d1, u1a, u1b, ub1, u2, ub2)


def kernel(edge_index, h, e, W1, b1, W2, b2, We, be, U1, ub1, U2, ub2):
    n, hdim = h.shape
    src2 = edge_index[0].reshape(-1, B)
    dst2 = edge_index[1].reshape(-1, B)
    w1a, w1b, w1c = W1[:hdim], W1[hdim:2 * hdim], W1[2 * hdim:]

    half = jnp.arange(hdim // 2)
    perm = jnp.stack([half, half + hdim // 2], axis=1).reshape(-1)
    hpa, hpb = _node_proj(h, w1a[:, perm], w1b[:, perm])
    hpa64 = jax.lax.bitcast_convert_type(
        hpa.reshape(n, hdim // 2, 2), jnp.int32)
    hpb64 = jax.lax.bitcast_convert_type(
        hpb.reshape(n, hdim // 2, 2), jnp.int32)
    g64 = _make_gather(e.shape[0])(hpa64, hpb64, src2, dst2)
    en, mw = _edge_mlp(g64, e, w1c, b1.reshape(1, -1), W2, b2.reshape(1, -1),
                       We.reshape(1, -1), be.reshape(1, 1))
    agg0, agg1, deg0, deg1 = _make_scatter(e.shape[0], n)(mw, dst2)
    h_new = _update_mlp(h, agg0, agg1, deg0, deg1, U1[:hdim], U1[hdim:],
                        ub1.reshape(1, -1), U2, ub2.reshape(1, -1))
    return (h_new, en)


# speedup vs baseline: 1.5752x; 1.2168x over previous
import functools

import jax
import jax.numpy as jnp
from jax import lax
from jax.experimental import pallas as pl
from jax.experimental.pallas import tpu as pltpu
from jax.experimental.pallas import tpu_sc as plsc

NC = 2
NS = 16
NW = NC * NS
B = 100
H = 128
C = 2
BLK = 2000


def _node_proj_body(h_ref, w1a_ref, w1b_ref, hpa_ref, hpb_ref):
    hblk = h_ref[...]
    hpa_ref[...] = jnp.dot(hblk, w1a_ref[...], preferred_element_type=jnp.float32)
    hpb_ref[...] = jnp.dot(hblk, w1b_ref[...], preferred_element_type=jnp.float32)


def _node_proj(h, w1a, w1b):
    n = h.shape[0]
    return pl.pallas_call(
        _node_proj_body,
        out_shape=(jax.ShapeDtypeStruct((n, H), jnp.float32),
                   jax.ShapeDtypeStruct((n, H), jnp.float32)),
        grid=(n // BLK,),
        in_specs=[pl.BlockSpec((BLK, H), lambda i: (i, 0)),
                  pl.BlockSpec((H, H), lambda i: (0, 0)),
                  pl.BlockSpec((H, H), lambda i: (0, 0))],
        out_specs=(pl.BlockSpec((BLK, H), lambda i: (i, 0)),
                   pl.BlockSpec((BLK, H), lambda i: (i, 0))),
    )(h, w1a, w1b)


def _make_gather(e_chunk):
    rows = e_chunk // B
    rpw = rows // NW
    mesh = plsc.VectorSubcoreMesh(core_axis_name="c", subcore_axis_name="s")

    @functools.partial(
        pl.kernel,
        out_type=jax.ShapeDtypeStruct((e_chunk, H), jnp.float32),
        mesh=mesh,
        compiler_params=pltpu.CompilerParams(use_tc_tiling_on_sc=False),
        scratch_types=[
            pltpu.VMEM((rpw, B), jnp.int32),
            pltpu.VMEM((rpw, B), jnp.int32),
            pltpu.VMEM((2, B, H), jnp.float32),
            pltpu.VMEM((2, B, H), jnp.float32),
            pltpu.SemaphoreType.DMA,
            pltpu.SemaphoreType.DMA,
            pltpu.SemaphoreType.DMA,
            pltpu.SemaphoreType.DMA,
        ],
    )
    def gather_k(hpa, hpb, src2, dst2, out, idxs, idxd, bufa, bufb,
                 sa0, sa1, sb0, sb1):
        cid = lax.axis_index("c")
        sid = lax.axis_index("s")
        wid = sid * NC + cid
        r0 = wid * rpw
        pltpu.sync_copy(src2.at[pl.ds(r0, rpw)], idxs)
        pltpu.sync_copy(dst2.at[pl.ds(r0, rpw)], idxd)
        sa = (sa0, sa1)
        sb = (sb0, sb1)

        def issue(ch, slot):
            pltpu.async_copy(hpa.at[idxs.at[ch]], bufa.at[slot], sa[slot])
            pltpu.async_copy(hpb.at[idxd.at[ch]], bufb.at[slot], sb[slot])

        def wait(slot):
            pltpu.make_async_copy(hpa.at[idxs.at[0]], bufa.at[slot], sa[slot]).wait()
            pltpu.make_async_copy(hpb.at[idxd.at[0]], bufb.at[slot], sb[slot]).wait()

        def process(slot, ch):
            def row(r, _):
                for k in range(H // 16):
                    s = pl.ds(k * 16, 16)
                    bufa[slot, r, s] = bufa[slot, r, s] + bufb[slot, r, s]
                return 0
            lax.fori_loop(0, B, row, 0)
            off = (r0 + ch) * B
            pltpu.sync_copy(bufa.at[slot], out.at[pl.ds(off, B)])

        issue(0, 0)

        def body2(i, _):
            ch0 = 2 * i
            wait(0)
            issue(ch0 + 1, 1)
            process(0, ch0)
            wait(1)

            @pl.when(ch0 + 2 < rpw)
            def _():
                issue(ch0 + 2, 0)

            process(1, ch0 + 1)
            return 0

        lax.fori_loop(0, rpw // 2, body2, 0)
        if rpw % 2:
            wait(0)
            process(0, rpw - 1)

    return gather_k


def _alloc_buf(e_total):
    def body(out_ref):
        out_ref[...] = jnp.zeros((8, H), jnp.float32)

    return pl.pallas_call(
        body,
        out_shape=jax.ShapeDtypeStruct((e_total, H), jnp.float32),
        grid=(1,),
        out_specs=pl.BlockSpec((8, H), lambda i: (0, 0)),
    )()


def _edge_body(g_ref, e_ref, en_in_ref, w1c_ref, b1_ref, w2_ref, b2_ref,
               wet_ref, be_ref, en_ref, mw_ref):
    del en_in_ref
    eblk = e_ref[...]
    x = (g_ref[...]
         + jnp.dot(eblk, w1c_ref[...], preferred_element_type=jnp.float32)
         + b1_ref[...])
    hid = x * jax.nn.sigmoid(x)
    msg = jnp.dot(hid, w2_ref[...], preferred_element_type=jnp.float32) + b2_ref[...]
    en = eblk + msg
    w = jax.nn.sigmoid(jnp.sum(en * wet_ref[...], axis=1, keepdims=True) + be_ref[...])
    en_ref[...] = en
    mw_ref[...] = msg * w


def _edge_mlp_chunk(c, gc, e, en_buf, w1c, b1, w2, b2, wet, be):
    e_total = e.shape[0]
    e_chunk = gc.shape[0]
    bpc = e_chunk // BLK
    wspec = pl.BlockSpec((H, H), lambda i: (0, 0))
    bspec = pl.BlockSpec((1, H), lambda i: (0, 0))
    return pl.pallas_call(
        _edge_body,
        out_shape=(jax.ShapeDtypeStruct((e_total, H), jnp.float32),
                   jax.ShapeDtypeStruct((e_chunk, H), jnp.float32)),
        grid=(bpc,),
        in_specs=[pl.BlockSpec((BLK, H), lambda i: (i, 0)),
                  pl.BlockSpec((BLK, H), lambda i, c=c, bpc=bpc: (i + c * bpc, 0)),
                  pl.BlockSpec(memory_space=pl.ANY),
                  wspec, bspec, wspec, bspec, bspec,
                  pl.BlockSpec((1, 1), lambda i: (0, 0))],
        out_specs=(pl.BlockSpec((BLK, H), lambda i, c=c, bpc=bpc: (i + c * bpc, 0)),
                   pl.BlockSpec((BLK, H), lambda i: (i, 0))),
        input_output_aliases={2: 0},
    )(gc, e, en_buf, w1c, b1, w2, b2, wet, be)


def _make_scatter(e_chunk, n):
    rows = e_chunk // B
    rpw = rows // NW
    mesh = plsc.VectorSubcoreMesh(core_axis_name="c", subcore_axis_name="s")
    f32 = jnp.float32

    @functools.partial(
        pl.kernel,
        out_type=(jax.ShapeDtypeStruct((n, H), f32),
                  jax.ShapeDtypeStruct((n, H), f32),
                  jax.ShapeDtypeStruct((n, 16), f32),
                  jax.ShapeDtypeStruct((n, 16), f32)),
        mesh=mesh,
        compiler_params=pltpu.CompilerParams(use_tc_tiling_on_sc=False),
        scratch_types=[
            pltpu.VMEM((rpw, B), jnp.int32),
            pltpu.VMEM((2, B, H), f32),
            pltpu.VMEM((B, 16), f32),
            pltpu.VMEM((B, 16), f32),
            pltpu.VMEM_SHARED((n, H), f32),
            pltpu.VMEM_SHARED((n, 16), f32),
            pltpu.SemaphoreType.DMA,
            pltpu.SemaphoreType.DMA,
        ],
    )
    def scatter_k(mw, dst2, agg0, agg1, deg0, deg1,
                  dstv, val, ones_v, dstage, aggs, degs, sv0, sv1):
        cid = lax.axis_index("c")
        sid = lax.axis_index("s")
        wid = sid * NC + cid
        r0 = wid * rpw
        nblk = n // B
        bpt = (nblk + NS - 1) // NS

        zero16 = jnp.zeros((16,), f32)
        one16 = jnp.ones((16,), f32)

        def zval(r, _):
            for k in range(H // 16):
                val[0, r, pl.ds(k * 16, 16)] = zero16
            return 0
        lax.fori_loop(0, B, zval, 0)

        def zsmall(r, _):
            dstage[r, pl.ds(0, 16)] = zero16
            ones_v[r, pl.ds(0, 16)] = one16
            return 0
        lax.fori_loop(0, B, zsmall, 0)

        def zblk(k, _):
            blk = k * NS + sid

            @pl.when(blk < nblk)
            def _():
                sl = pl.ds(blk * B, B)
                pltpu.sync_copy(val.at[0], aggs.at[sl])
                pltpu.sync_copy(dstage, degs.at[sl])
            return 0
        lax.fori_loop(0, bpt, zblk, 0)
        plsc.subcore_barrier()

        pltpu.sync_copy(dst2.at[pl.ds(r0, rpw)], dstv)
        sv = (sv0, sv1)

        def issue(ch, slot):
            off = (r0 + ch) * B
            pltpu.async_copy(mw.at[pl.ds(off, B)], val.at[slot], sv[slot])

        def wait(slot):
            pltpu.make_async_copy(mw.at[pl.ds(0, B)], val.at[slot], sv[slot]).wait()

        def process(slot, ch):
            pltpu.sync_copy(val.at[slot], aggs.at[dstv.at[ch]], add=True)
            pltpu.sync_copy(ones_v, degs.at[dstv.at[ch]], add=True)

        issue(0, 0)

        def body2(i, _):
            ch0 = 2 * i
            wait(0)
            issue(ch0 + 1, 1)
            process(0, ch0)
            wait(1)

            @pl.when(ch0 + 2 < rpw)
            def _():
                issue(ch0 + 2, 0)

            process(1, ch0 + 1)
            return 0

        lax.fori_loop(0, rpw // 2, body2, 0)
        if rpw % 2:
            wait(0)
            process(0, rpw - 1)
        plsc.subcore_barrier()

        def wblk(k, _):
            blk = k * NS + sid

            @pl.when(blk < nblk)
            def _():
                sl = pl.ds(blk * B, B)
                pltpu.sync_copy(aggs.at[sl], val.at[0])
                pltpu.sync_copy(degs.at[sl], dstage)

                @pl.when(cid == 0)
                def _():
                    pltpu.sync_copy(val.at[0], agg0.at[sl])
                    pltpu.sync_copy(dstage, deg0.at[sl])

                @pl.when(cid == 1)
                def _():
                    pltpu.sync_copy(val.at[0], agg1.at[sl])
                    pltpu.sync_copy(dstage, deg1.at[sl])
            return 0
        lax.fori_loop(0, bpt, wblk, 0)

    return scatter_k


def _make_update_body(nparts):
    def body(*refs):
        h_ref = refs[0]
        ps = refs[1:1 + nparts]
        ds = refs[1 + nparts:1 + 2 * nparts]
        u1a_ref, u1b_ref, ub1_ref, u2_ref, ub2_ref, out_ref = refs[1 + 2 * nparts:]
        s = ps[0][...]
        for p in ps[1:]:
            s = s + p[...]
        d = ds[0][...][:, 0:1]
        for dd in ds[1:]:
            d = d + dd[...][:, 0:1]
        agg = s / (d + 1e-6)
        x = (jnp.dot(h_ref[...], u1a_ref[...], preferred_element_type=jnp.float32)
             + jnp.dot(agg, u1b_ref[...], preferred_element_type=jnp.float32)
             + ub1_ref[...])
        hid = x * jax.nn.sigmoid(x)
        out_ref[...] = (jnp.dot(hid, u2_ref[...], preferred_element_type=jnp.float32)
                        + ub2_ref[...])
    return body


def _update_mlp(h, aggs, degs, u1a, u1b, ub1, u2, ub2):
    n = h.shape[0]
    nparts = len(aggs)
    wspec = pl.BlockSpec((H, H), lambda i: (0, 0))
    bspec = pl.BlockSpec((1, H), lambda i: (0, 0))
    nspec = pl.BlockSpec((BLK, H), lambda i: (i, 0))
    dspec = pl.BlockSpec((BLK, 16), lambda i: (i, 0))
    return pl.pallas_call(
        _make_update_body(nparts),
        out_shape=jax.ShapeDtypeStruct((n, H), jnp.float32),
        grid=(n // BLK,),
        in_specs=([nspec] + [nspec] * nparts + [dspec] * nparts
                  + [wspec, wspec, bspec, wspec, bspec]),
        out_specs=nspec,
    )(h, *aggs, *degs, u1a, u1b, ub1, u2, ub2)


def kernel(edge_index, h, e, W1, b1, W2, b2, We, be, U1, ub1, U2, ub2):
    n, hdim = h.shape
    e_total = e.shape[0]
    e_chunk = e_total // C
    rows_per_chunk = e_chunk // B
    src2 = edge_index[0].reshape(-1, B)
    dst2 = edge_index[1].reshape(-1, B)
    w1a, w1b, w1c = W1[:hdim], W1[hdim:2 * hdim], W1[2 * hdim:]
    b1r, b2r = b1.reshape(1, -1), b2.reshape(1, -1)
    wet, ber = We.reshape(1, -1), be.reshape(1, 1)

    hpa, hpb = _node_proj(h, w1a, w1b)
    gather = _make_gather(e_chunk)
    scatter = _make_scatter(e_chunk, n)

    gs = []
    for c in range(C):
        sl = slice(c * rows_per_chunk, (c + 1) * rows_per_chunk)
        gs.append(gather(hpa, hpb, src2[sl], dst2[sl]))

    en_buf = _alloc_buf(e_total)
    agg_parts, deg_parts = [], []
    for c in range(C):
        sl = slice(c * rows_per_chunk, (c + 1) * rows_per_chunk)
        en_buf, mw = _edge_mlp_chunk(c, gs[c], e, en_buf, w1c, b1r, W2, b2r,
                                     wet, ber)
        a0, a1, d0, d1 = scatter(mw, dst2[sl])
        agg_parts += [a0, a1]
        deg_parts += [d0, d1]

    h_new = _update_mlp(h, agg_parts, deg_parts, U1[:hdim], U1[hdim:],
                        ub1.reshape(1, -1), U2, ub2.reshape(1, -1))
    return (h_new, en_buf)


# speedup vs baseline: 1.6957x; 1.0765x over previous
import functools

import jax
import jax.numpy as jnp
from jax import lax
from jax.experimental import pallas as pl
from jax.experimental.pallas import tpu as pltpu
from jax.experimental.pallas import tpu_sc as plsc

NC = 2
NS = 16
NW = NC * NS
B = 100
H = 128
C = 4
BLK = 2000


def _node_proj_body(h_ref, w1a_ref, w1b_ref, hpa_ref, hpb_ref):
    hblk = h_ref[...]
    hpa_ref[...] = jnp.dot(hblk, w1a_ref[...], preferred_element_type=jnp.float32)
    hpb_ref[...] = jnp.dot(hblk, w1b_ref[...], preferred_element_type=jnp.float32)


def _node_proj(h, w1a, w1b):
    n = h.shape[0]
    return pl.pallas_call(
        _node_proj_body,
        out_shape=(jax.ShapeDtypeStruct((n, H), jnp.float32),
                   jax.ShapeDtypeStruct((n, H), jnp.float32)),
        grid=(n // BLK,),
        in_specs=[pl.BlockSpec((BLK, H), lambda i: (i, 0)),
                  pl.BlockSpec((H, H), lambda i: (0, 0)),
                  pl.BlockSpec((H, H), lambda i: (0, 0))],
        out_specs=(pl.BlockSpec((BLK, H), lambda i: (i, 0)),
                   pl.BlockSpec((BLK, H), lambda i: (i, 0))),
    )(h, w1a, w1b)


def _make_gather(e_chunk):
    rows = e_chunk // B
    rpw = rows // NW
    mesh = plsc.VectorSubcoreMesh(core_axis_name="c", subcore_axis_name="s")

    @functools.partial(
        pl.kernel,
        out_type=jax.ShapeDtypeStruct((e_chunk, H), jnp.float32),
        mesh=mesh,
        compiler_params=pltpu.CompilerParams(use_tc_tiling_on_sc=False),
        scratch_types=[
            pltpu.VMEM((rpw, B), jnp.int32),
            pltpu.VMEM((rpw, B), jnp.int32),
            pltpu.VMEM((2, B, H), jnp.float32),
            pltpu.VMEM((2, B, H), jnp.float32),
            pltpu.SemaphoreType.DMA,
            pltpu.SemaphoreType.DMA,
            pltpu.SemaphoreType.DMA,
            pltpu.SemaphoreType.DMA,
        ],
    )
    def gather_k(hpa, hpb, src2, dst2, out, idxs, idxd, bufa, bufb,
                 sa0, sa1, sb0, sb1):
        cid = lax.axis_index("c")
        sid = lax.axis_index("s")
        wid = sid * NC + cid
        r0 = wid * rpw
        pltpu.sync_copy(src2.at[pl.ds(r0, rpw)], idxs)
        pltpu.sync_copy(dst2.at[pl.ds(r0, rpw)], idxd)
        sa = (sa0, sa1)
        sb = (sb0, sb1)

        def issue(ch, slot):
            pltpu.async_copy(hpa.at[idxs.at[ch]], bufa.at[slot], sa[slot])
            pltpu.async_copy(hpb.at[idxd.at[ch]], bufb.at[slot], sb[slot])

        def wait(slot):
            pltpu.make_async_copy(hpa.at[idxs.at[0]], bufa.at[slot], sa[slot]).wait()
            pltpu.make_async_copy(hpb.at[idxd.at[0]], bufb.at[slot], sb[slot]).wait()

        def process(slot, ch):
            def row(r, _):
                for k in range(H // 16):
                    s = pl.ds(k * 16, 16)
                    bufa[slot, r, s] = bufa[slot, r, s] + bufb[slot, r, s]
                return 0
            lax.fori_loop(0, B, row, 0)
            off = (r0 + ch) * B
            pltpu.sync_copy(bufa.at[slot], out.at[pl.ds(off, B)])

        issue(0, 0)

        def body2(i, _):
            ch0 = 2 * i
            wait(0)
            issue(ch0 + 1, 1)
            process(0, ch0)
            wait(1)

            @pl.when(ch0 + 2 < rpw)
            def _():
                issue(ch0 + 2, 0)

            process(1, ch0 + 1)
            return 0

        lax.fori_loop(0, rpw // 2, body2, 0)
        if rpw % 2:
            wait(0)
            process(0, rpw - 1)

    return gather_k


def _alloc_buf(e_total):
    def body(out_ref):
        out_ref[...] = jnp.zeros((8, H), jnp.float32)

    return pl.pallas_call(
        body,
        out_shape=jax.ShapeDtypeStruct((e_total, H), jnp.float32),
        grid=(1,),
        out_specs=pl.BlockSpec((8, H), lambda i: (0, 0)),
    )()


def _edge_body(g_ref, e_ref, en_in_ref, w1c_ref, b1_ref, w2_ref, b2_ref,
               wet_ref, be_ref, en_ref, mw_ref):
    del en_in_ref
    eblk = e_ref[...]
    x = (g_ref[...]
         + jnp.dot(eblk, w1c_ref[...], preferred_element_type=jnp.float32)
         + b1_ref[...])
    hid = x * jax.nn.sigmoid(x)
    msg = jnp.dot(hid, w2_ref[...], preferred_element_type=jnp.float32) + b2_ref[...]
    en = eblk + msg
    w = jax.nn.sigmoid(jnp.sum(en * wet_ref[...], axis=1, keepdims=True) + be_ref[...])
    en_ref[...] = en
    mw_ref[...] = msg * w


def _edge_mlp_chunk(c, gc, e, en_buf, w1c, b1, w2, b2, wet, be):
    e_total = e.shape[0]
    e_chunk = gc.shape[0]
    bpc = e_chunk // BLK
    wspec = pl.BlockSpec((H, H), lambda i: (0, 0))
    bspec = pl.BlockSpec((1, H), lambda i: (0, 0))
    return pl.pallas_call(
        _edge_body,
        out_shape=(jax.ShapeDtypeStruct((e_total, H), jnp.float32),
                   jax.ShapeDtypeStruct((e_chunk, H), jnp.float32)),
        grid=(bpc,),
        in_specs=[pl.BlockSpec((BLK, H), lambda i: (i, 0)),
                  pl.BlockSpec((BLK, H), lambda i, c=c, bpc=bpc: (i + c * bpc, 0)),
                  pl.BlockSpec(memory_space=pl.ANY),
                  wspec, bspec, wspec, bspec, bspec,
                  pl.BlockSpec((1, 1), lambda i: (0, 0))],
        out_specs=(pl.BlockSpec((BLK, H), lambda i, c=c, bpc=bpc: (i + c * bpc, 0)),
                   pl.BlockSpec((BLK, H), lambda i: (i, 0))),
        input_output_aliases={2: 0},
    )(gc, e, en_buf, w1c, b1, w2, b2, wet, be)


def _make_scatter(e_chunk, n):
    rows = e_chunk // B
    rpw = rows // NW
    mesh = plsc.VectorSubcoreMesh(core_axis_name="c", subcore_axis_name="s")
    f32 = jnp.float32

    @functools.partial(
        pl.kernel,
        out_type=(jax.ShapeDtypeStruct((n, H), f32),
                  jax.ShapeDtypeStruct((n, H), f32),
                  jax.ShapeDtypeStruct((n, 16), f32),
                  jax.ShapeDtypeStruct((n, 16), f32)),
        mesh=mesh,
        compiler_params=pltpu.CompilerParams(use_tc_tiling_on_sc=False),
        scratch_types=[
            pltpu.VMEM((rpw, B), jnp.int32),
            pltpu.VMEM((2, B, H), f32),
            pltpu.VMEM((B, 16), f32),
            pltpu.VMEM((B, 16), f32),
            pltpu.VMEM_SHARED((n, H), f32),
            pltpu.VMEM_SHARED((n, 16), f32),
            pltpu.SemaphoreType.DMA,
            pltpu.SemaphoreType.DMA,
        ],
    )
    def scatter_k(mw, dst2, agg0, agg1, deg0, deg1,
                  dstv, val, ones_v, dstage, aggs, degs, sv0, sv1):
        cid = lax.axis_index("c")
        sid = lax.axis_index("s")
        wid = sid * NC + cid
        r0 = wid * rpw
        nblk = n // B
        bpt = (nblk + NS - 1) // NS

        zero16 = jnp.zeros((16,), f32)
        one16 = jnp.ones((16,), f32)

        def zval(r, _):
            for k in range(H // 16):
                val[0, r, pl.ds(k * 16, 16)] = zero16
            return 0
        lax.fori_loop(0, B, zval, 0)

        def zsmall(r, _):
            dstage[r, pl.ds(0, 16)] = zero16
            ones_v[r, pl.ds(0, 16)] = one16
            return 0
        lax.fori_loop(0, B, zsmall, 0)

        def zblk(k, _):
            blk = k * NS + sid

            @pl.when(blk < nblk)
            def _():
                sl = pl.ds(blk * B, B)
                pltpu.sync_copy(val.at[0], aggs.at[sl])
                pltpu.sync_copy(dstage, degs.at[sl])
            return 0
        lax.fori_loop(0, bpt, zblk, 0)
        plsc.subcore_barrier()

        pltpu.sync_copy(dst2.at[pl.ds(r0, rpw)], dstv)
        sv = (sv0, sv1)

        def issue(ch, slot):
            off = (r0 + ch) * B
            pltpu.async_copy(mw.at[pl.ds(off, B)], val.at[slot], sv[slot])

        def wait(slot):
            pltpu.make_async_copy(mw.at[pl.ds(0, B)], val.at[slot], sv[slot]).wait()

        def process(slot, ch):
            pltpu.sync_copy(val.at[slot], aggs.at[dstv.at[ch]], add=True)
            pltpu.sync_copy(ones_v, degs.at[dstv.at[ch]], add=True)

        issue(0, 0)

        def body2(i, _):
            ch0 = 2 * i
            wait(0)
            issue(ch0 + 1, 1)
            process(0, ch0)
            wait(1)

            @pl.when(ch0 + 2 < rpw)
            def _():
                issue(ch0 + 2, 0)

            process(1, ch0 + 1)
            return 0

        lax.fori_loop(0, rpw // 2, body2, 0)
        if rpw % 2:
            wait(0)
            process(0, rpw - 1)
        plsc.subcore_barrier()

        def wblk(k, _):
            blk = k * NS + sid

            @pl.when(blk < nblk)
            def _():
                sl = pl.ds(blk * B, B)
                pltpu.sync_copy(aggs.at[sl], val.at[0])
                pltpu.sync_copy(degs.at[sl], dstage)

                @pl.when(cid == 0)
                def _():
                    pltpu.sync_copy(val.at[0], agg0.at[sl])
                    pltpu.sync_copy(dstage, deg0.at[sl])

                @pl.when(cid == 1)
                def _():
                    pltpu.sync_copy(val.at[0], agg1.at[sl])
                    pltpu.sync_copy(dstage, deg1.at[sl])
            return 0
        lax.fori_loop(0, bpt, wblk, 0)

    return scatter_k


def _make_update_body(nparts):
    def body(*refs):
        h_ref = refs[0]
        ps = refs[1:1 + nparts]
        ds = refs[1 + nparts:1 + 2 * nparts]
        u1a_ref, u1b_ref, ub1_ref, u2_ref, ub2_ref, out_ref = refs[1 + 2 * nparts:]
        s = ps[0][...]
        for p in ps[1:]:
            s = s + p[...]
        d = ds[0][...][:, 0:1]
        for dd in ds[1:]:
            d = d + dd[...][:, 0:1]
        agg = s / (d + 1e-6)
        x = (jnp.dot(h_ref[...], u1a_ref[...], preferred_element_type=jnp.float32)
             + jnp.dot(agg, u1b_ref[...], preferred_element_type=jnp.float32)
             + ub1_ref[...])
        hid = x * jax.nn.sigmoid(x)
        out_ref[...] = (jnp.dot(hid, u2_ref[...], preferred_element_type=jnp.float32)
                        + ub2_ref[...])
    return body


def _update_mlp(h, aggs, degs, u1a, u1b, ub1, u2, ub2):
    n = h.shape[0]
    nparts = len(aggs)
    wspec = pl.BlockSpec((H, H), lambda i: (0, 0))
    bspec = pl.BlockSpec((1, H), lambda i: (0, 0))
    nspec = pl.BlockSpec((BLK, H), lambda i: (i, 0))
    dspec = pl.BlockSpec((BLK, 16), lambda i: (i, 0))
    return pl.pallas_call(
        _make_update_body(nparts),
        out_shape=jax.ShapeDtypeStruct((n, H), jnp.float32),
        grid=(n // BLK,),
        in_specs=([nspec] + [nspec] * nparts + [dspec] * nparts
                  + [wspec, wspec, bspec, wspec, bspec]),
        out_specs=nspec,
    )(h, *aggs, *degs, u1a, u1b, ub1, u2, ub2)


def kernel(edge_index, h, e, W1, b1, W2, b2, We, be, U1, ub1, U2, ub2):
    n, hdim = h.shape
    e_total = e.shape[0]
    e_chunk = e_total // C
    rows_per_chunk = e_chunk // B
    src2 = edge_index[0].reshape(-1, B)
    dst2 = edge_index[1].reshape(-1, B)
    w1a, w1b, w1c = W1[:hdim], W1[hdim:2 * hdim], W1[2 * hdim:]
    b1r, b2r = b1.reshape(1, -1), b2.reshape(1, -1)
    wet, ber = We.reshape(1, -1), be.reshape(1, 1)

    hpa, hpb = _node_proj(h, w1a, w1b)
    gather = _make_gather(e_chunk)
    scatter = _make_scatter(e_chunk, n)

    gs = []
    for c in range(C):
        sl = slice(c * rows_per_chunk, (c + 1) * rows_per_chunk)
        gs.append(gather(hpa, hpb, src2[sl], dst2[sl]))

    en_buf = _alloc_buf(e_total)
    agg_parts, deg_parts = [], []
    for c in range(C):
        sl = slice(c * rows_per_chunk, (c + 1) * rows_per_chunk)
        en_buf, mw = _edge_mlp_chunk(c, gs[c], e, en_buf, w1c, b1r, W2, b2r,
                                     wet, ber)
        a0, a1, d0, d1 = scatter(mw, dst2[sl])
        agg_parts += [a0, a1]
        deg_parts += [d0, d1]

    h_new = _update_mlp(h, agg_parts, deg_parts, U1[:hdim], U1[hdim:],
                        ub1.reshape(1, -1), U2, ub2.reshape(1, -1))
    return (h_new, en_buf)


# speedup vs baseline: 1.8231x; 1.0751x over previous
import functools

import jax
import jax.numpy as jnp
from jax import lax
from jax.experimental import pallas as pl
from jax.experimental.pallas import tpu as pltpu
from jax.experimental.pallas import tpu_sc as plsc

NC = 2
NS = 16
NW = NC * NS
B = 100
H = 128
C = 4
BLK = 2000


def _node_proj_body(h_ref, w1a_ref, w1b_ref, hpa_ref, hpb_ref):
    hblk = h_ref[...]
    hpa_ref[...] = jnp.dot(hblk, w1a_ref[...], preferred_element_type=jnp.float32)
    hpb_ref[...] = jnp.dot(hblk, w1b_ref[...], preferred_element_type=jnp.float32)


def _node_proj(h, w1a, w1b):
    n = h.shape[0]
    return pl.pallas_call(
        _node_proj_body,
        out_shape=(jax.ShapeDtypeStruct((n, H), jnp.float32),
                   jax.ShapeDtypeStruct((n, H), jnp.float32)),
        grid=(n // BLK,),
        in_specs=[pl.BlockSpec((BLK, H), lambda i: (i, 0)),
                  pl.BlockSpec((H, H), lambda i: (0, 0)),
                  pl.BlockSpec((H, H), lambda i: (0, 0))],
        out_specs=(pl.BlockSpec((BLK, H), lambda i: (i, 0)),
                   pl.BlockSpec((BLK, H), lambda i: (i, 0))),
    )(h, w1a, w1b)


def _make_gather(e_chunk):
    rows = e_chunk // B
    rpw = rows // NW
    mesh = plsc.VectorSubcoreMesh(core_axis_name="c", subcore_axis_name="s")

    @functools.partial(
        pl.kernel,
        out_type=jax.ShapeDtypeStruct((e_chunk, H), jnp.float32),
        mesh=mesh,
        compiler_params=pltpu.CompilerParams(use_tc_tiling_on_sc=False),
        scratch_types=[
            pltpu.VMEM((rpw, B), jnp.int32),
            pltpu.VMEM((rpw, B), jnp.int32),
            pltpu.VMEM((2, B, H), jnp.float32),
            pltpu.VMEM((2, B, H), jnp.float32),
            pltpu.SemaphoreType.DMA,
            pltpu.SemaphoreType.DMA,
            pltpu.SemaphoreType.DMA,
            pltpu.SemaphoreType.DMA,
        ],
    )
    def gather_k(hpa, hpb, src2, dst2, out, idxs, idxd, bufa, bufb,
                 sa0, sa1, sb0, sb1):
        cid = lax.axis_index("c")
        sid = lax.axis_index("s")
        wid = sid * NC + cid
        r0 = wid * rpw
        pltpu.sync_copy(src2.at[pl.ds(r0, rpw)], idxs)
        pltpu.sync_copy(dst2.at[pl.ds(r0, rpw)], idxd)
        sa = (sa0, sa1)
        sb = (sb0, sb1)

        def issue(ch, slot):
            pltpu.async_copy(hpa.at[idxs.at[ch]], bufa.at[slot], sa[slot])
            pltpu.async_copy(hpb.at[idxd.at[ch]], bufb.at[slot], sb[slot])

        def wait(slot):
            pltpu.make_async_copy(hpa.at[idxs.at[0]], bufa.at[slot], sa[slot]).wait()
            pltpu.make_async_copy(hpb.at[idxd.at[0]], bufb.at[slot], sb[slot]).wait()

        def process(slot, ch):
            def row(r, _):
                for k in range(H // 16):
                    s = pl.ds(k * 16, 16)
                    bufa[slot, r, s] = bufa[slot, r, s] + bufb[slot, r, s]
                return 0
            lax.fori_loop(0, B, row, 0)
            off = (r0 + ch) * B
            pltpu.sync_copy(bufa.at[slot], out.at[pl.ds(off, B)])

        issue(0, 0)

        def body2(i, _):
            ch0 = 2 * i
            wait(0)
            issue(ch0 + 1, 1)
            process(0, ch0)
            wait(1)

            @pl.when(ch0 + 2 < rpw)
            def _():
                issue(ch0 + 2, 0)

            process(1, ch0 + 1)
            return 0

        lax.fori_loop(0, rpw // 2, body2, 0)
        if rpw % 2:
            wait(0)
            process(0, rpw - 1)

    return gather_k


def _alloc_buf(e_total):
    def body(out_ref):
        out_ref[...] = jnp.zeros((8, H), jnp.float32)

    return pl.pallas_call(
        body,
        out_shape=jax.ShapeDtypeStruct((e_total, H), jnp.float32),
        grid=(1,),
        out_specs=pl.BlockSpec((8, H), lambda i: (0, 0)),
    )()


def _edge_body(g_ref, e_ref, en_in_ref, w1c_ref, b1_ref, w2_ref, b2_ref,
               we_ref, w2we_ref, wb_ref, en_ref, mw_ref):
    del en_in_ref
    eblk = e_ref[...]
    we1 = jnp.dot(eblk, we_ref[...], preferred_element_type=jnp.float32)
    x = (g_ref[...]
         + jnp.dot(eblk, w1c_ref[...], preferred_element_type=jnp.float32)
         + b1_ref[...])
    hid = x * jax.nn.sigmoid(x)
    we2 = jnp.dot(hid, w2we_ref[...], preferred_element_type=jnp.float32)
    msg = jnp.dot(hid, w2_ref[...], preferred_element_type=jnp.float32) + b2_ref[...]
    w = jax.nn.sigmoid(we1 + we2 + wb_ref[...])
    en_ref[...] = eblk + msg
    mw_ref[...] = msg * w


def _edge_mlp_chunk(c, gc, e, en_buf, w1c, b1, w2, b2, we, w2we, wb):
    e_total = e.shape[0]
    e_chunk = gc.shape[0]
    bpc = e_chunk // BLK
    wspec = pl.BlockSpec((H, H), lambda i: (0, 0))
    bspec = pl.BlockSpec((1, H), lambda i: (0, 0))
    cspec = pl.BlockSpec((H, 1), lambda i: (0, 0))
    return pl.pallas_call(
        _edge_body,
        out_shape=(jax.ShapeDtypeStruct((e_total, H), jnp.float32),
                   jax.ShapeDtypeStruct((e_chunk, H), jnp.float32)),
        grid=(bpc,),
        in_specs=[pl.BlockSpec((BLK, H), lambda i: (i, 0)),
                  pl.BlockSpec((BLK, H), lambda i, c=c, bpc=bpc: (i + c * bpc, 0)),
                  pl.BlockSpec(memory_space=pl.ANY),
                  wspec, bspec, wspec, bspec, cspec, cspec,
                  pl.BlockSpec((1, 1), lambda i: (0, 0))],
        out_specs=(pl.BlockSpec((BLK, H), lambda i, c=c, bpc=bpc: (i + c * bpc, 0)),
                   pl.BlockSpec((BLK, H), lambda i: (i, 0))),
        input_output_aliases={2: 0},
    )(gc, e, en_buf, w1c, b1, w2, b2, we, w2we, wb)


def _make_scatter(e_chunk, n):
    rows = e_chunk // B
    rpw = rows // NW
    mesh = plsc.VectorSubcoreMesh(core_axis_name="c", subcore_axis_name="s")
    f32 = jnp.float32

    @functools.partial(
        pl.kernel,
        out_type=(jax.ShapeDtypeStruct((n, H), f32),
                  jax.ShapeDtypeStruct((n, H), f32),
                  jax.ShapeDtypeStruct((n, 16), f32),
                  jax.ShapeDtypeStruct((n, 16), f32)),
        mesh=mesh,
        compiler_params=pltpu.CompilerParams(use_tc_tiling_on_sc=False),
        scratch_types=[
            pltpu.VMEM((rpw, B), jnp.int32),
            pltpu.VMEM((2, B, H), f32),
            pltpu.VMEM((B, 16), f32),
            pltpu.VMEM((B, 16), f32),
            pltpu.VMEM_SHARED((n, H), f32),
            pltpu.VMEM_SHARED((n, 16), f32),
            pltpu.SemaphoreType.DMA,
            pltpu.SemaphoreType.DMA,
        ],
    )
    def scatter_k(mw, dst2, agg0, agg1, deg0, deg1,
                  dstv, val, ones_v, dstage, aggs, degs, sv0, sv1):
        cid = lax.axis_index("c")
        sid = lax.axis_index("s")
        wid = sid * NC + cid
        r0 = wid * rpw
        nblk = n // B
        bpt = (nblk + NS - 1) // NS

        zero16 = jnp.zeros((16,), f32)
        one16 = jnp.ones((16,), f32)

        def zval(r, _):
            for k in range(H // 16):
                val[0, r, pl.ds(k * 16, 16)] = zero16
            return 0
        lax.fori_loop(0, B, zval, 0)

        def zsmall(r, _):
            dstage[r, pl.ds(0, 16)] = zero16
            ones_v[r, pl.ds(0, 16)] = one16
            return 0
        lax.fori_loop(0, B, zsmall, 0)

        def zblk(k, _):
            blk = k * NS + sid

            @pl.when(blk < nblk)
            def _():
                sl = pl.ds(blk * B, B)
                pltpu.sync_copy(val.at[0], aggs.at[sl])
                pltpu.sync_copy(dstage, degs.at[sl])
            return 0
        lax.fori_loop(0, bpt, zblk, 0)
        plsc.subcore_barrier()

        pltpu.sync_copy(dst2.at[pl.ds(r0, rpw)], dstv)
        sv = (sv0, sv1)

        def issue(ch, slot):
            off = (r0 + ch) * B
            pltpu.async_copy(mw.at[pl.ds(off, B)], val.at[slot], sv[slot])

        def wait(slot):
            pltpu.make_async_copy(mw.at[pl.ds(0, B)], val.at[slot], sv[slot]).wait()

        def process(slot, ch):
            pltpu.sync_copy(val.at[slot], aggs.at[dstv.at[ch]], add=True)
            pltpu.sync_copy(ones_v, degs.at[dstv.at[ch]], add=True)

        issue(0, 0)

        def body2(i, _):
            ch0 = 2 * i
            wait(0)
            issue(ch0 + 1, 1)
            process(0, ch0)
            wait(1)

            @pl.when(ch0 + 2 < rpw)
            def _():
                issue(ch0 + 2, 0)

            process(1, ch0 + 1)
            return 0

        lax.fori_loop(0, rpw // 2, body2, 0)
        if rpw % 2:
            wait(0)
            process(0, rpw - 1)
        plsc.subcore_barrier()

        def wblk(k, _):
            blk = k * NS + sid

            @pl.when(blk < nblk)
            def _():
                sl = pl.ds(blk * B, B)
                pltpu.sync_copy(aggs.at[sl], val.at[0])
                pltpu.sync_copy(degs.at[sl], dstage)

                @pl.when(cid == 0)
                def _():
                    pltpu.sync_copy(val.at[0], agg0.at[sl])
                    pltpu.sync_copy(dstage, deg0.at[sl])

                @pl.when(cid == 1)
                def _():
                    pltpu.sync_copy(val.at[0], agg1.at[sl])
                    pltpu.sync_copy(dstage, deg1.at[sl])
            return 0
        lax.fori_loop(0, bpt, wblk, 0)

    return scatter_k


def _make_update_body(nparts):
    def body(*refs):
        h_ref = refs[0]
        ps = refs[1:1 + nparts]
        ds = refs[1 + nparts:1 + 2 * nparts]
        u1a_ref, u1b_ref, ub1_ref, u2_ref, ub2_ref, out_ref = refs[1 + 2 * nparts:]
        s = ps[0][...]
        for p in ps[1:]:
            s = s + p[...]
        d = ds[0][...][:, 0:1]
        for dd in ds[1:]:
            d = d + dd[...][:, 0:1]
        agg = s / (d + 1e-6)
        x = (jnp.dot(h_ref[...], u1a_ref[...], preferred_element_type=jnp.float32)
             + jnp.dot(agg, u1b_ref[...], preferred_element_type=jnp.float32)
             + ub1_ref[...])
        hid = x * jax.nn.sigmoid(x)
        out_ref[...] = (jnp.dot(hid, u2_ref[...], preferred_element_type=jnp.float32)
                        + ub2_ref[...])
    return body


def _update_mlp(h, aggs, degs, u1a, u1b, ub1, u2, ub2):
    n = h.shape[0]
    nparts = len(aggs)
    wspec = pl.BlockSpec((H, H), lambda i: (0, 0))
    bspec = pl.BlockSpec((1, H), lambda i: (0, 0))
    nspec = pl.BlockSpec((BLK, H), lambda i: (i, 0))
    dspec = pl.BlockSpec((BLK, 16), lambda i: (i, 0))
    return pl.pallas_call(
        _make_update_body(nparts),
        out_shape=jax.ShapeDtypeStruct((n, H), jnp.float32),
        grid=(n // BLK,),
        in_specs=([nspec] + [nspec] * nparts + [dspec] * nparts
                  + [wspec, wspec, bspec, wspec, bspec]),
        out_specs=nspec,
    )(h, *aggs, *degs, u1a, u1b, ub1, u2, ub2)


def kernel(edge_index, h, e, W1, b1, W2, b2, We, be, U1, ub1, U2, ub2):
    n, hdim = h.shape
    e_total = e.shape[0]
    e_chunk = e_total // C
    rows_per_chunk = e_chunk // B
    src2 = edge_index[0].reshape(-1, B)
    dst2 = edge_index[1].reshape(-1, B)
    w1a, w1b, w1c = W1[:hdim], W1[hdim:2 * hdim], W1[2 * hdim:]
    b1r, b2r = b1.reshape(1, -1), b2.reshape(1, -1)
    w2we = W2 @ We
    wb = (b2 @ We + be).reshape(1, 1)

    hpa, hpb = _node_proj(h, w1a, w1b)
    gather = _make_gather(e_chunk)
    scatter = _make_scatter(e_chunk, n)

    gs = []
    for c in range(C):
        sl = slice(c * rows_per_chunk, (c + 1) * rows_per_chunk)
        gs.append(gather(hpa, hpb, src2[sl], dst2[sl]))

    en_buf = _alloc_buf(e_total)
    agg_parts, deg_parts = [], []
    for c in range(C):
        sl = slice(c * rows_per_chunk, (c + 1) * rows_per_chunk)
        en_buf, mw = _edge_mlp_chunk(c, gs[c], e, en_buf, w1c, b1r, W2, b2r,
                                     We, w2we, wb)
        a0, a1, d0, d1 = scatter(mw, dst2[sl])
        agg_parts += [a0, a1]
        deg_parts += [d0, d1]

    h_new = _update_mlp(h, agg_parts, deg_parts, U1[:hdim], U1[hdim:],
                        ub1.reshape(1, -1), U2, ub2.reshape(1, -1))
    return (h_new, en_buf)


# speedup vs baseline: 1.9382x; 1.0632x over previous
import functools

import jax
import jax.numpy as jnp
from jax import lax
from jax.experimental import pallas as pl
from jax.experimental.pallas import tpu as pltpu
from jax.experimental.pallas import tpu_sc as plsc

NC = 2
NS = 16
NW = NC * NS
B = 100
H = 128
C = 4
BLK = 2000


def _node_proj_body(h_ref, w1a_ref, w1b_ref, hpa_ref, hpb_ref):
    hblk = h_ref[...]
    hpa_ref[...] = jnp.dot(hblk, w1a_ref[...], preferred_element_type=jnp.float32)
    hpb_ref[...] = jnp.dot(hblk, w1b_ref[...], preferred_element_type=jnp.float32)


def _node_proj(h, w1a, w1b):
    n = h.shape[0]
    return pl.pallas_call(
        _node_proj_body,
        out_shape=(jax.ShapeDtypeStruct((n, H), jnp.float32),
                   jax.ShapeDtypeStruct((n, H), jnp.float32)),
        grid=(n // BLK,),
        in_specs=[pl.BlockSpec((BLK, H), lambda i: (i, 0)),
                  pl.BlockSpec((H, H), lambda i: (0, 0)),
                  pl.BlockSpec((H, H), lambda i: (0, 0))],
        out_specs=(pl.BlockSpec((BLK, H), lambda i: (i, 0)),
                   pl.BlockSpec((BLK, H), lambda i: (i, 0))),
    )(h, w1a, w1b)


def _make_gather(e_chunk):
    rows = e_chunk // B
    rpw = rows // NW
    mesh = plsc.VectorSubcoreMesh(core_axis_name="c", subcore_axis_name="s")

    @functools.partial(
        pl.kernel,
        out_type=jax.ShapeDtypeStruct((e_chunk, H), jnp.float32),
        mesh=mesh,
        compiler_params=pltpu.CompilerParams(use_tc_tiling_on_sc=False),
        scratch_types=[
            pltpu.VMEM((rpw, B), jnp.int32),
            pltpu.VMEM((rpw, B), jnp.int32),
            pltpu.VMEM((2, B, H), jnp.float32),
            pltpu.VMEM((2, B, H), jnp.float32),
            pltpu.SemaphoreType.DMA,
            pltpu.SemaphoreType.DMA,
            pltpu.SemaphoreType.DMA,
            pltpu.SemaphoreType.DMA,
        ],
    )
    def gather_k(hpa, hpb, src2, dst2, out, idxs, idxd, bufa, bufb,
                 sa0, sa1, sb0, sb1):
        cid = lax.axis_index("c")
        sid = lax.axis_index("s")
        wid = sid * NC + cid
        r0 = wid * rpw
        pltpu.sync_copy(src2.at[pl.ds(r0, rpw)], idxs)
        pltpu.sync_copy(dst2.at[pl.ds(r0, rpw)], idxd)
        sa = (sa0, sa1)
        sb = (sb0, sb1)

        def issue(ch, slot):
            pltpu.async_copy(hpa.at[idxs.at[ch]], bufa.at[slot], sa[slot])
            pltpu.async_copy(hpb.at[idxd.at[ch]], bufb.at[slot], sb[slot])

        def wait(slot):
            pltpu.make_async_copy(hpa.at[idxs.at[0]], bufa.at[slot], sa[slot]).wait()
            pltpu.make_async_copy(hpb.at[idxd.at[0]], bufb.at[slot], sb[slot]).wait()

        def process(slot, ch):
            def row(r, _):
                for k in range(H // 16):
                    s = pl.ds(k * 16, 16)
                    bufa[slot, r, s] = bufa[slot, r, s] + bufb[slot, r, s]
                return 0
            lax.fori_loop(0, B, row, 0)
            off = (r0 + ch) * B
            pltpu.sync_copy(bufa.at[slot], out.at[pl.ds(off, B)])

        issue(0, 0)

        def body2(i, _):
            ch0 = 2 * i
            wait(0)
            issue(ch0 + 1, 1)
            process(0, ch0)
            wait(1)

            @pl.when(ch0 + 2 < rpw)
            def _():
                issue(ch0 + 2, 0)

            process(1, ch0 + 1)
            return 0

        lax.fori_loop(0, rpw // 2, body2, 0)
        if rpw % 2:
            wait(0)
            process(0, rpw - 1)

    return gather_k


def _alloc_buf(e_total):
    def body(out_ref):
        out_ref[...] = jnp.zeros((8, H), jnp.float32)

    return pl.pallas_call(
        body,
        out_shape=jax.ShapeDtypeStruct((e_total, H), jnp.float32),
        grid=(1,),
        out_specs=pl.BlockSpec((8, H), lambda i: (0, 0)),
    )()


def _edge_body(g_ref, e_ref, en_in_ref, w1c_ref, b1_ref, w2_ref, b2_ref,
               we_ref, w2we_ref, wb_ref, en_ref, mw_ref):
    del en_in_ref
    eblk = e_ref[...]
    we1 = jnp.dot(eblk, we_ref[...], preferred_element_type=jnp.float32)
    x = (g_ref[...]
         + jnp.dot(eblk, w1c_ref[...], preferred_element_type=jnp.float32)
         + b1_ref[...])
    hid = x * jax.nn.sigmoid(x)
    we2 = jnp.dot(hid, w2we_ref[...], preferred_element_type=jnp.float32)
    msg = jnp.dot(hid, w2_ref[...], preferred_element_type=jnp.float32) + b2_ref[...]
    w = jax.nn.sigmoid(we1 + we2 + wb_ref[...])
    en_ref[...] = eblk + msg
    mw_ref[...] = msg * w


def _edge_mlp_chunk(c, gc, e, en_buf, w1c, b1, w2, b2, we, w2we, wb):
    e_total = e.shape[0]
    e_chunk = gc.shape[0]
    bpc = e_chunk // BLK
    wspec = pl.BlockSpec((H, H), lambda i: (0, 0))
    bspec = pl.BlockSpec((1, H), lambda i: (0, 0))
    cspec = pl.BlockSpec((H, 1), lambda i: (0, 0))
    return pl.pallas_call(
        _edge_body,
        out_shape=(jax.ShapeDtypeStruct((e_total, H), jnp.float32),
                   jax.ShapeDtypeStruct((e_chunk, H), jnp.float32)),
        grid=(bpc,),
        in_specs=[pl.BlockSpec((BLK, H), lambda i: (i, 0)),
                  pl.BlockSpec((BLK, H), lambda i, c=c, bpc=bpc: (i + c * bpc, 0)),
                  pl.BlockSpec(memory_space=pl.ANY),
                  wspec, bspec, wspec, bspec, cspec, cspec,
                  pl.BlockSpec((1, 1), lambda i: (0, 0))],
        out_specs=(pl.BlockSpec((BLK, H), lambda i, c=c, bpc=bpc: (i + c * bpc, 0)),
                   pl.BlockSpec((BLK, H), lambda i: (i, 0))),
        input_output_aliases={2: 0},
    )(gc, e, en_buf, w1c, b1, w2, b2, we, w2we, wb)


def _make_scatter(e_chunk, n, nmw):
    rows = e_chunk // B
    rpw = rows // NW
    mesh = plsc.VectorSubcoreMesh(core_axis_name="c", subcore_axis_name="s")
    f32 = jnp.float32

    @functools.partial(
        pl.kernel,
        out_type=(jax.ShapeDtypeStruct((n, H), f32),
                  jax.ShapeDtypeStruct((n, H), f32),
                  jax.ShapeDtypeStruct((n, 16), f32),
                  jax.ShapeDtypeStruct((n, 16), f32)),
        mesh=mesh,
        compiler_params=pltpu.CompilerParams(use_tc_tiling_on_sc=False),
        scratch_types=[
            pltpu.VMEM((nmw * rpw, B), jnp.int32),
            pltpu.VMEM((2, B, H), f32),
            pltpu.VMEM((B, 16), f32),
            pltpu.VMEM((B, 16), f32),
            pltpu.VMEM_SHARED((n, H), f32),
            pltpu.VMEM_SHARED((n, 16), f32),
            pltpu.SemaphoreType.DMA,
            pltpu.SemaphoreType.DMA,
        ],
    )
    def scatter_k(*args):
        mws = args[:nmw]
        (dst2, agg0, agg1, deg0, deg1,
         dstv, val, ones_v, dstage, aggs, degs, sv0, sv1) = args[nmw:]
        cid = lax.axis_index("c")
        sid = lax.axis_index("s")
        wid = sid * NC + cid
        r0 = wid * rpw
        nblk = n // B
        bpt = (nblk + NS - 1) // NS

        zero16 = jnp.zeros((16,), f32)
        one16 = jnp.ones((16,), f32)

        def zval(r, _):
            for k in range(H // 16):
                val[0, r, pl.ds(k * 16, 16)] = zero16
            return 0
        lax.fori_loop(0, B, zval, 0)

        def zsmall(r, _):
            dstage[r, pl.ds(0, 16)] = zero16
            ones_v[r, pl.ds(0, 16)] = one16
            return 0
        lax.fori_loop(0, B, zsmall, 0)

        def zblk(k, _):
            blk = k * NS + sid

            @pl.when(blk < nblk)
            def _():
                sl = pl.ds(blk * B, B)
                pltpu.sync_copy(val.at[0], aggs.at[sl])
                pltpu.sync_copy(dstage, degs.at[sl])
            return 0
        lax.fori_loop(0, bpt, zblk, 0)
        plsc.subcore_barrier()

        for m in range(nmw):
            pltpu.sync_copy(dst2.at[pl.ds(m * rows + r0, rpw)],
                            dstv.at[pl.ds(m * rpw, rpw)])
        sv = (sv0, sv1)

        for m in range(nmw):
            mw = mws[m]

            def issue(ch, slot, mw=mw):
                off = (r0 + ch) * B
                pltpu.async_copy(mw.at[pl.ds(off, B)], val.at[slot], sv[slot])

            def wait(slot, mw=mw):
                pltpu.make_async_copy(mw.at[pl.ds(0, B)], val.at[slot],
                                      sv[slot]).wait()

            def process(slot, ch, m=m):
                pltpu.sync_copy(val.at[slot], aggs.at[dstv.at[m * rpw + ch]],
                                add=True)
                pltpu.sync_copy(ones_v, degs.at[dstv.at[m * rpw + ch]],
                                add=True)

            issue(0, 0)

            def body2(i, _, issue=issue, wait=wait, process=process):
                ch0 = 2 * i
                wait(0)
                issue(ch0 + 1, 1)
                process(0, ch0)
                wait(1)

                @pl.when(ch0 + 2 < rpw)
                def _():
                    issue(ch0 + 2, 0)

                process(1, ch0 + 1)
                return 0

            lax.fori_loop(0, rpw // 2, body2, 0)
            if rpw % 2:
                wait(0)
                process(0, rpw - 1)
        plsc.subcore_barrier()

        def wblk(k, _):
            blk = k * NS + sid

            @pl.when(blk < nblk)
            def _():
                sl = pl.ds(blk * B, B)
                pltpu.sync_copy(aggs.at[sl], val.at[0])
                pltpu.sync_copy(degs.at[sl], dstage)

                @pl.when(cid == 0)
                def _():
                    pltpu.sync_copy(val.at[0], agg0.at[sl])
                    pltpu.sync_copy(dstage, deg0.at[sl])

                @pl.when(cid == 1)
                def _():
                    pltpu.sync_copy(val.at[0], agg1.at[sl])
                    pltpu.sync_copy(dstage, deg1.at[sl])
            return 0
        lax.fori_loop(0, bpt, wblk, 0)

    return scatter_k


def _make_update_body(nparts):
    def body(*refs):
        h_ref = refs[0]
        ps = refs[1:1 + nparts]
        ds = refs[1 + nparts:1 + 2 * nparts]
        u1a_ref, u1b_ref, ub1_ref, u2_ref, ub2_ref, out_ref = refs[1 + 2 * nparts:]
        s = ps[0][...]
        for p in ps[1:]:
            s = s + p[...]
        d = ds[0][...][:, 0:1]
        for dd in ds[1:]:
            d = d + dd[...][:, 0:1]
        agg = s / (d + 1e-6)
        x = (jnp.dot(h_ref[...], u1a_ref[...], preferred_element_type=jnp.float32)
             + jnp.dot(agg, u1b_ref[...], preferred_element_type=jnp.float32)
             + ub1_ref[...])
        hid = x * jax.nn.sigmoid(x)
        out_ref[...] = (jnp.dot(hid, u2_ref[...], preferred_element_type=jnp.float32)
                        + ub2_ref[...])
    return body


def _update_mlp(h, aggs, degs, u1a, u1b, ub1, u2, ub2):
    n = h.shape[0]
    nparts = len(aggs)
    wspec = pl.BlockSpec((H, H), lambda i: (0, 0))
    bspec = pl.BlockSpec((1, H), lambda i: (0, 0))
    nspec = pl.BlockSpec((BLK, H), lambda i: (i, 0))
    dspec = pl.BlockSpec((BLK, 16), lambda i: (i, 0))
    return pl.pallas_call(
        _make_update_body(nparts),
        out_shape=jax.ShapeDtypeStruct((n, H), jnp.float32),
        grid=(n // BLK,),
        in_specs=([nspec] + [nspec] * nparts + [dspec] * nparts
                  + [wspec, wspec, bspec, wspec, bspec]),
        out_specs=nspec,
    )(h, *aggs, *degs, u1a, u1b, ub1, u2, ub2)


def kernel(edge_index, h, e, W1, b1, W2, b2, We, be, U1, ub1, U2, ub2):
    n, hdim = h.shape
    e_total = e.shape[0]
    e_chunk = e_total // C
    rows_per_chunk = e_chunk // B
    src2 = edge_index[0].reshape(-1, B)
    dst2 = edge_index[1].reshape(-1, B)
    w1a, w1b, w1c = W1[:hdim], W1[hdim:2 * hdim], W1[2 * hdim:]
    b1r, b2r = b1.reshape(1, -1), b2.reshape(1, -1)
    w2we = W2 @ We
    wb = (b2 @ We + be).reshape(1, 1)

    hpa, hpb = _node_proj(h, w1a, w1b)
    gather = _make_gather(e_chunk)

    gs = []
    for c in range(C):
        sl = slice(c * rows_per_chunk, (c + 1) * rows_per_chunk)
        gs.append(gather(hpa, hpb, src2[sl], dst2[sl]))

    en_buf = _alloc_buf(e_total)
    mws = []
    for c in range(C):
        en_buf, mw = _edge_mlp_chunk(c, gs[c], e, en_buf, w1c, b1r, W2, b2r,
                                     We, w2we, wb)
        mws.append(mw)

    agg_parts, deg_parts = [], []
    for ms, me in ((0, C - 1), (C - 1, C)):
        nmw = me - ms
        sl = slice(ms * rows_per_chunk, me * rows_per_chunk)
        a0, a1, d0, d1 = _make_scatter(e_chunk, n, nmw)(*mws[ms:me], dst2[sl])
        agg_parts += [a0, a1]
        deg_parts += [d0, d1]

    h_new = _update_mlp(h, agg_parts, deg_parts, U1[:hdim], U1[hdim:],
                        ub1.reshape(1, -1), U2, ub2.reshape(1, -1))
    return (h_new, en_buf)
